# Initial kernel scaffold; baseline (speedup 1.0000x reference)
import jax
import jax.numpy as jnp
from jax.experimental import pallas as pl


def kernel(dem, ts0, ts1, ts2, ts3, W_dem1, b_dem1, W_dem2, b_dem2, W_lin, b_lin):
    raise NotImplementedError("write your pallas kernel here")


# speedup vs baseline: 7.0494x; 7.0494x over previous
import jax
import jax.numpy as jnp
from jax import lax
from jax.experimental import pallas as pl
from jax.experimental.pallas import tpu as pltpu

_N, _L = 32, 2048
_C = 256
_TL = 256
_NBLK = _L // _TL
_DEM_IN, _DEM_H, _DEM_OUT = 8, 40, 20
_LIN_IN = 4 * _C + _DEM_OUT
_OUT_C = 3 * _C + _DEM_OUT
_EPS = 1e-5
_NEG = float("-inf")


def _sdown(v, k, fill):
    pad = jnp.full((k, v.shape[1]), fill, v.dtype)
    return jnp.concatenate([pad, v[: v.shape[0] - k, :]], axis=0)


def _maxscan(v, fill):
    k = 1
    while k < v.shape[0]:
        v = jnp.maximum(v, _sdown(v, k, fill))
        k *= 2
    return v


def _bdot(a, b):
    return jnp.dot(a.astype(jnp.bfloat16), b.astype(jnp.bfloat16),
                   preferred_element_type=jnp.float32)


def _ln(x):
    mu = jnp.mean(x, axis=-1, keepdims=True)
    xc = x - mu
    var = jnp.mean(xc * xc, axis=-1, keepdims=True)
    return xc * lax.rsqrt(var + _EPS)


def _body(dem_ref, ts0_ref, ts1_ref, ts2_ref, ts3_ref,
          wd1_ref, bd1_ref, wd2_ref, bd2_ref, wlin_ref, blin_ref, tri_ref,
          out_ref, ti_ref, act_ref,
          cs_ref, cm_ref, ca_ref):
    n = pl.program_id(0)
    j = pl.program_id(1)

    @pl.when(j == 0)
    def _init():
        cs_ref[...] = jnp.zeros_like(cs_ref)
        cm_ref[...] = jnp.full_like(cm_ref, _NEG)
        ca_ref[...] = jnp.full_like(ca_ref, -1)

    h = jnp.maximum(_bdot(dem_ref[...], wd1_ref[...]) + bd1_ref[...], 0.0)
    dall = jnp.maximum(_bdot(h, wd2_ref[...]) + bd2_ref[...], 0.0)
    rows = lax.broadcasted_iota(jnp.int32, (_N, _DEM_OUT), 0)
    d_row = jnp.sum(jnp.where(rows == n, dall, 0.0), axis=0, keepdims=True)

    c_row = _bdot(d_row, wlin_ref[4 * _C:, :]) + blin_ref[...]
    x = (_bdot(ts0_ref[0], wlin_ref[0 * _C:1 * _C, :])
         + _bdot(ts1_ref[0], wlin_ref[1 * _C:2 * _C, :])
         + _bdot(ts2_ref[0], wlin_ref[2 * _C:3 * _C, :])
         + _bdot(ts3_ref[0], wlin_ref[3 * _C:4 * _C, :])
         + c_row)

    csum = lax.dot_general(tri_ref[...], x, (((1,), (0,)), ((), ())),
                           precision=lax.Precision.HIGHEST,
                           preferred_element_type=jnp.float32)
    csum = csum + cs_ref[0:1, :]
    cs_ref[...] = jnp.broadcast_to(csum[_TL - 1:_TL, :], (8, _C))

    cm = cm_ref[0:1, :]
    m_loc = _maxscan(x, _NEG)
    m_glob = jnp.maximum(m_loc, cm)
    e = jnp.maximum(_sdown(m_loc, 1, _NEG), cm)
    cm_ref[...] = jnp.broadcast_to(m_glob[_TL - 1:_TL, :], (8, _C))

    tmat = j * _TL + lax.broadcasted_iota(jnp.int32, (_TL, _C), 0)
    s = jnp.where(x > e, tmat, -1)
    carg = jnp.maximum(_maxscan(s, -1), ca_ref[0:1, :])
    ca_ref[...] = jnp.broadcast_to(carg[_TL - 1:_TL, :], (8, _C))

    pad_mask = tmat < (_L - 1)
    p_max = jnp.where(pad_mask, jnp.maximum(m_glob, 0.0), m_glob)
    ti = jnp.where(pad_mask & (m_glob <= 0.0), tmat - (_L - 1), carg)

    tf = (tmat + 1).astype(jnp.float32)
    p_avg = csum * jnp.float32(1.0 / _L)
    p_sum = csum * lax.rsqrt(tf)

    out_ref[0, :, 0 * _C:1 * _C] = _ln(p_max)
    out_ref[0, :, 1 * _C:2 * _C] = _ln(p_avg)
    out_ref[0, :, 2 * _C:3 * _C] = _ln(p_sum)
    out_ref[0, :, 3 * _C:] = jnp.broadcast_to(d_row, (_TL, _DEM_OUT))
    ti_ref[0] = ti.T
    act_ref[0] = p_max.T


def kernel(dem, ts0, ts1, ts2, ts3, W_dem1, b_dem1, W_dem2, b_dem2, W_lin, b_lin):
    tri = jnp.tril(jnp.ones((_TL, _TL), jnp.float32))
    full = lambda shape: pl.BlockSpec(shape, lambda n, j: (0,) * len(shape))
    grid = (_N, _NBLK)
    out, ti, act = pl.pallas_call(
        _body,
        grid=grid,
        in_specs=[
            full((_N, _DEM_IN)),
            pl.BlockSpec((1, _TL, _C), lambda n, j: (n, j, 0)),
            pl.BlockSpec((1, _TL, _C), lambda n, j: (n, j, 0)),
            pl.BlockSpec((1, _TL, _C), lambda n, j: (n, j, 0)),
            pl.BlockSpec((1, _TL, _C), lambda n, j: (n, j, 0)),
            full((_DEM_IN, _DEM_H)),
            full((1, _DEM_H)),
            full((_DEM_H, _DEM_OUT)),
            full((1, _DEM_OUT)),
            full((_LIN_IN, _C)),
            full((1, _C)),
            full((_TL, _TL)),
        ],
        out_specs=[
            pl.BlockSpec((1, _TL, _OUT_C), lambda n, j: (n, j, 0)),
            pl.BlockSpec((1, _C, _TL), lambda n, j: (n, 0, j)),
            pl.BlockSpec((1, _C, _TL), lambda n, j: (n, 0, j)),
        ],
        out_shape=[
            jax.ShapeDtypeStruct((_N, _L, _OUT_C), jnp.float32),
            jax.ShapeDtypeStruct((_N, _C, _L), jnp.int32),
            jax.ShapeDtypeStruct((_N, _C, _L), jnp.float32),
        ],
        scratch_shapes=[
            pltpu.VMEM((8, _C), jnp.float32),
            pltpu.VMEM((8, _C), jnp.float32),
            pltpu.VMEM((8, _C), jnp.int32),
        ],
        compiler_params=pltpu.CompilerParams(
            dimension_semantics=("parallel", "arbitrary"),
        ),
        name="deep_pool_encoder",
    )(dem, ts0, ts1, ts2, ts3,
      W_dem1, b_dem1.reshape(1, _DEM_H), W_dem2, b_dem2.reshape(1, _DEM_OUT),
      W_lin, b_lin.reshape(1, _C), tri)
    return out, ti, act


# speedup vs baseline: 11.8187x; 1.6766x over previous
import jax
import jax.numpy as jnp
from jax import lax
from jax.experimental import pallas as pl
from jax.experimental.pallas import tpu as pltpu

_N, _L = 32, 2048
_C = 256
_B = 8
_NB1 = _N // _B
_TLB = 128
_NJ = _L // _TLB
_DEM_IN, _DEM_H, _DEM_OUT = 8, 40, 20
_LIN_IN = 4 * _C + _DEM_OUT
_OUT_C = 3 * _C + _DEM_OUT
_EPS = 1e-5
_NEG = float("-inf")


def _sdown(v, k, fill):
    pad = jnp.full((_B, k, _C), fill, v.dtype)
    return jnp.concatenate([pad, v[:, : _TLB - k, :]], axis=1)


def _maxscan(v, fill):
    k = 1
    while k < _TLB:
        v = jnp.maximum(v, _sdown(v, k, fill))
        k *= 2
    return v


def _bdot(a, b):
    return jnp.dot(a.astype(jnp.bfloat16), b.astype(jnp.bfloat16),
                   preferred_element_type=jnp.float32)


def _body(dem_ref, ts0_ref, ts1_ref, ts2_ref, ts3_ref,
          wd1_ref, bd1_ref, wd2_ref, bd2_ref, wlin_ref, blin_ref, tri_ref,
          out_ref, ti_ref, act_ref,
          cs_ref, cm_ref, ca_ref):
    n1 = pl.program_id(0)
    j = pl.program_id(1)

    @pl.when(j == 0)
    def _init():
        cs_ref[...] = jnp.zeros_like(cs_ref)
        cm_ref[...] = jnp.full_like(cm_ref, _NEG)
        ca_ref[...] = jnp.full_like(ca_ref, -1.0)

    h = jnp.maximum(_bdot(dem_ref[...], wd1_ref[...]) + bd1_ref[...], 0.0)
    dall = jnp.maximum(_bdot(h, wd2_ref[...]) + bd2_ref[...], 0.0)
    d4 = dall.reshape(_NB1, _B, _DEM_OUT)
    gmask = lax.broadcasted_iota(jnp.int32, (_NB1, _B, _DEM_OUT), 0) == n1
    d8 = jnp.sum(jnp.where(gmask, d4, 0.0), axis=0)

    t0 = ts0_ref[...].reshape(_B * _TLB, _C)
    t1 = ts1_ref[...].reshape(_B * _TLB, _C)
    t2 = ts2_ref[...].reshape(_B * _TLB, _C)
    t3 = ts3_ref[...].reshape(_B * _TLB, _C)
    xw = (_bdot(t0, wlin_ref[0 * _C:1 * _C, :])
          + _bdot(t1, wlin_ref[1 * _C:2 * _C, :])
          + _bdot(t2, wlin_ref[2 * _C:3 * _C, :])
          + _bdot(t3, wlin_ref[3 * _C:4 * _C, :]))
    cdem = _bdot(d8, wlin_ref[4 * _C:, :]) + blin_ref[...]
    x3 = xw.reshape(_B, _TLB, _C) + cdem.reshape(_B, 1, _C)
    x2 = x3.reshape(_B * _TLB, _C)

    xhi = x2.astype(jnp.bfloat16)
    xlo = (x2 - xhi.astype(jnp.float32)).astype(jnp.bfloat16)
    trib = tri_ref[...]
    cs_c = cs_ref[...]
    chunks, carries = [], []
    for b in range(_B):
        sl = slice(b * _TLB, (b + 1) * _TLB)
        ck = (jnp.dot(trib, xhi[sl], preferred_element_type=jnp.float32)
              + jnp.dot(trib, xlo[sl], preferred_element_type=jnp.float32)
              + cs_c[b:b + 1, :])
        chunks.append(ck)
        carries.append(ck[_TLB - 1:_TLB, :])
    csum3 = jnp.concatenate(chunks, axis=0).reshape(_B, _TLB, _C)
    cs_ref[...] = jnp.concatenate(carries, axis=0)

    cmc = cm_ref[...].reshape(_B, 1, _C)
    m_loc = _maxscan(x3, _NEG)
    m_glob = jnp.maximum(m_loc, cmc)
    e = jnp.maximum(_sdown(m_loc, 1, _NEG), cmc)
    cm_ref[...] = m_glob[:, _TLB - 1, :]

    tf = (j * _TLB
          + lax.broadcasted_iota(jnp.int32, (_B, _TLB, _C), 1)
          ).astype(jnp.float32)
    s = jnp.where(x3 > e, tf, -1.0)
    carg = jnp.maximum(_maxscan(s, -1.0), ca_ref[...].reshape(_B, 1, _C))
    ca_ref[...] = carg[:, _TLB - 1, :]

    csT = jnp.swapaxes(csum3, 1, 2)
    mT = jnp.swapaxes(m_glob, 1, 2)
    caT = jnp.swapaxes(carg, 1, 2)

    tT = j * _TLB + lax.broadcasted_iota(jnp.int32, (_B, _C, _TLB), 2)
    padT = tT < (_L - 1)
    pmaxT = jnp.where(padT, jnp.maximum(mT, 0.0), mT)
    tiT = jnp.where(padT & (pmaxT <= 0.0), tT - (_L - 1), caT.astype(jnp.int32))

    muc = jnp.mean(csT, axis=1, keepdims=True)
    varc = jnp.mean(csT * csT, axis=1, keepdims=True) - muc * muc
    dcT = csT - muc
    tlane = (j * _TLB
             + lax.broadcasted_iota(jnp.int32, (_B, 1, _TLB), 2)
             ).astype(jnp.float32)
    ln_avg = dcT * lax.rsqrt(varc + jnp.float32(_EPS * _L * _L))
    ln_sum = dcT * lax.rsqrt(varc + _EPS * (tlane + 1.0))

    mum = jnp.mean(pmaxT, axis=1, keepdims=True)
    varm = jnp.mean(pmaxT * pmaxT, axis=1, keepdims=True) - mum * mum
    ln_max = (pmaxT - mum) * lax.rsqrt(varm + _EPS)

    act_ref[...] = pmaxT
    ti_ref[...] = tiT
    d8T = d8.T
    for b in range(_B):
        out_ref[0 * _C:1 * _C, 0, 0, b, :] = ln_max[b]
        out_ref[1 * _C:2 * _C, 0, 0, b, :] = ln_avg[b]
        out_ref[2 * _C:3 * _C, 0, 0, b, :] = ln_sum[b]
        out_ref[3 * _C:, 0, 0, b, :] = jnp.broadcast_to(
            d8T[:, b:b + 1], (_DEM_OUT, _TLB))


def kernel(dem, ts0, ts1, ts2, ts3, W_dem1, b_dem1, W_dem2, b_dem2, W_lin, b_lin):
    tri = jnp.tril(jnp.ones((_TLB, _TLB), jnp.float32)).astype(jnp.bfloat16)
    full = lambda shape: pl.BlockSpec(shape, lambda n1, j: (0,) * len(shape))
    ts_spec = pl.BlockSpec((_B, _TLB, _C), lambda n1, j: (n1, j, 0))
    o5, ti, act = pl.pallas_call(
        _body,
        grid=(_NB1, _NJ),
        in_specs=[
            full((_N, _DEM_IN)),
            ts_spec, ts_spec, ts_spec, ts_spec,
            full((_DEM_IN, _DEM_H)),
            full((1, _DEM_H)),
            full((_DEM_H, _DEM_OUT)),
            full((1, _DEM_OUT)),
            full((_LIN_IN, _C)),
            full((1, _C)),
            full((_TLB, _TLB)),
        ],
        out_specs=[
            pl.BlockSpec((_OUT_C, 1, 1, _B, _TLB),
                         lambda n1, j: (0, n1, j, 0, 0)),
            pl.BlockSpec((_B, _C, _TLB), lambda n1, j: (n1, 0, j)),
            pl.BlockSpec((_B, _C, _TLB), lambda n1, j: (n1, 0, j)),
        ],
        out_shape=[
            jax.ShapeDtypeStruct((_OUT_C, _NB1, _NJ, _B, _TLB), jnp.float32),
            jax.ShapeDtypeStruct((_N, _C, _L), jnp.int32),
            jax.ShapeDtypeStruct((_N, _C, _L), jnp.float32),
        ],
        scratch_shapes=[
            pltpu.VMEM((_B, _C), jnp.float32),
            pltpu.VMEM((_B, _C), jnp.float32),
            pltpu.VMEM((_B, _C), jnp.float32),
        ],
        compiler_params=pltpu.CompilerParams(
            dimension_semantics=("parallel", "arbitrary"),
        ),
        name="deep_pool_encoder",
    )(dem, ts0, ts1, ts2, ts3,
      W_dem1, b_dem1.reshape(1, _DEM_H), W_dem2, b_dem2.reshape(1, _DEM_OUT),
      W_lin, b_lin.reshape(1, _C), tri)
    out = o5.transpose(1, 3, 2, 4, 0).reshape(_N, _L, _OUT_C)
    return out, ti, act


# speedup vs baseline: 19.8699x; 1.6812x over previous
import jax
import jax.numpy as jnp
from jax import lax
from jax.experimental import pallas as pl
from jax.experimental.pallas import tpu as pltpu

_N, _L = 32, 2048
_C = 256
_B = 8
_NB1 = _N // _B
_TLB = 128
_NJ = _L // _TLB
_NIT = _NB1 * _NJ
_DEM_IN, _DEM_H, _DEM_OUT = 8, 40, 20
_LIN_IN = 4 * _C + _DEM_OUT
_OUT_C = 3 * _C + _DEM_OUT
_EPS = 1e-5
_NEG = float("-inf")


def _sdown(v, k, fill):
    pad = jnp.full((_B, k, _C), fill, v.dtype)
    return jnp.concatenate([pad, v[:, : _TLB - k, :]], axis=1)


def _maxscan(v, fill):
    k = 1
    while k < _TLB:
        v = jnp.maximum(v, _sdown(v, k, fill))
        k *= 2
    return v


def _bdot(a, b):
    return jnp.dot(a.astype(jnp.bfloat16), b.astype(jnp.bfloat16),
                   preferred_element_type=jnp.float32)


def _body(dem_ref, ts0_ref, ts1_ref, ts2_ref, ts3_ref,
          wd1_ref, bd1_ref, wd2_ref, bd2_ref, wlin_ref, blin_ref, tri_ref,
          out_hbm, ti_ref, act_ref,
          cs_ref, cm_ref, ca_ref, obuf, osem):
    n1 = pl.program_id(0)
    j = pl.program_id(1)
    it = n1 * _NJ + j
    slot = lax.rem(it, 2)

    def owait(sl):
        for b in range(_B):
            pltpu.make_async_copy(obuf.at[sl, b], obuf.at[sl, b],
                                  osem.at[sl]).wait()

    @pl.when(it >= 2)
    def _reclaim():
        owait(slot)

    @pl.when(j == 0)
    def _init():
        cs_ref[...] = jnp.zeros_like(cs_ref)
        cm_ref[...] = jnp.full_like(cm_ref, _NEG)
        ca_ref[...] = jnp.full_like(ca_ref, -1.0)

    h = jnp.maximum(_bdot(dem_ref[...], wd1_ref[...]) + bd1_ref[...], 0.0)
    dall = jnp.maximum(_bdot(h, wd2_ref[...]) + bd2_ref[...], 0.0)
    d4 = dall.reshape(_NB1, _B, _DEM_OUT)
    gmask = lax.broadcasted_iota(jnp.int32, (_NB1, _B, _DEM_OUT), 0) == n1
    d8 = jnp.sum(jnp.where(gmask, d4, 0.0), axis=0)

    t0 = ts0_ref[...].reshape(_B * _TLB, _C)
    t1 = ts1_ref[...].reshape(_B * _TLB, _C)
    t2 = ts2_ref[...].reshape(_B * _TLB, _C)
    t3 = ts3_ref[...].reshape(_B * _TLB, _C)
    xw = (_bdot(t0, wlin_ref[0 * _C:1 * _C, :])
          + _bdot(t1, wlin_ref[1 * _C:2 * _C, :])
          + _bdot(t2, wlin_ref[2 * _C:3 * _C, :])
          + _bdot(t3, wlin_ref[3 * _C:4 * _C, :]))
    cdem = _bdot(d8, wlin_ref[4 * _C:, :]) + blin_ref[...]
    x3 = xw.reshape(_B, _TLB, _C) + cdem.reshape(_B, 1, _C)
    x2 = x3.reshape(_B * _TLB, _C)

    xhi = x2.astype(jnp.bfloat16)
    xlo = (x2 - xhi.astype(jnp.float32)).astype(jnp.bfloat16)
    trib = tri_ref[...]
    cs_c = cs_ref[...]
    chunks, carries = [], []
    for b in range(_B):
        sl = slice(b * _TLB, (b + 1) * _TLB)
        ck = (jnp.dot(trib, xhi[sl], preferred_element_type=jnp.float32)
              + jnp.dot(trib, xlo[sl], preferred_element_type=jnp.float32)
              + cs_c[b:b + 1, :])
        chunks.append(ck)
        carries.append(ck[_TLB - 1:_TLB, :])
    csum3 = jnp.concatenate(chunks, axis=0).reshape(_B, _TLB, _C)
    cs_ref[...] = jnp.concatenate(carries, axis=0)

    cmc = cm_ref[...].reshape(_B, 1, _C)
    m_loc = _maxscan(x3, _NEG)
    m_glob = jnp.maximum(m_loc, cmc)
    e = jnp.maximum(_sdown(m_loc, 1, _NEG), cmc)
    cm_ref[...] = m_glob[:, _TLB - 1, :]

    tf = (j * _TLB
          + lax.broadcasted_iota(jnp.int32, (_B, _TLB, _C), 1)
          ).astype(jnp.float32)
    s = jnp.where(x3 > e, tf, -1.0)
    carg = jnp.maximum(_maxscan(s, -1.0), ca_ref[...].reshape(_B, 1, _C))
    ca_ref[...] = carg[:, _TLB - 1, :]

    csT = jnp.swapaxes(csum3, 1, 2)
    mT = jnp.swapaxes(m_glob, 1, 2)
    caT = jnp.swapaxes(carg, 1, 2)

    tT = j * _TLB + lax.broadcasted_iota(jnp.int32, (_B, _C, _TLB), 2)
    padT = tT < (_L - 1)
    pmaxT = jnp.where(padT, jnp.maximum(mT, 0.0), mT)
    tiT = jnp.where(padT & (pmaxT <= 0.0), tT - (_L - 1), caT.astype(jnp.int32))

    muc = jnp.mean(csT, axis=1, keepdims=True)
    varc = jnp.mean(csT * csT, axis=1, keepdims=True) - muc * muc
    dcT = csT - muc
    tlane = (j * _TLB
             + lax.broadcasted_iota(jnp.int32, (_B, 1, _TLB), 2)
             ).astype(jnp.float32)
    ln_avg = dcT * lax.rsqrt(varc + jnp.float32(_EPS * _L * _L))
    ln_sum = dcT * lax.rsqrt(varc + _EPS * (tlane + 1.0))

    mum = jnp.mean(pmaxT, axis=1, keepdims=True)
    varm = jnp.mean(pmaxT * pmaxT, axis=1, keepdims=True) - mum * mum
    ln_max = (pmaxT - mum) * lax.rsqrt(varm + _EPS)

    act_ref[...] = pmaxT
    ti_ref[...] = tiT

    obuf[slot, :, 0 * _C:1 * _C, :] = ln_max
    obuf[slot, :, 1 * _C:2 * _C, :] = ln_avg
    obuf[slot, :, 2 * _C:3 * _C, :] = ln_sum
    d8T = d8.T
    for b in range(_B):
        obuf[slot, b, 3 * _C:, :] = jnp.broadcast_to(
            d8T[:, b:b + 1], (_DEM_OUT, _TLB))
    for b in range(_B):
        pltpu.make_async_copy(obuf.at[slot, b],
                              out_hbm.at[:, n1, j, b, :],
                              osem.at[slot]).start()

    @pl.when(it == _NIT - 1)
    def _drain():
        owait(1 - slot)
        owait(slot)


def kernel(dem, ts0, ts1, ts2, ts3, W_dem1, b_dem1, W_dem2, b_dem2, W_lin, b_lin):
    tri = jnp.tril(jnp.ones((_TLB, _TLB), jnp.float32)).astype(jnp.bfloat16)
    full = lambda shape: pl.BlockSpec(shape, lambda n1, j: (0,) * len(shape))
    ts_spec = pl.BlockSpec((_B, _TLB, _C), lambda n1, j: (n1, j, 0))
    o5, ti, act = pl.pallas_call(
        _body,
        grid=(_NB1, _NJ),
        in_specs=[
            full((_N, _DEM_IN)),
            ts_spec, ts_spec, ts_spec, ts_spec,
            full((_DEM_IN, _DEM_H)),
            full((1, _DEM_H)),
            full((_DEM_H, _DEM_OUT)),
            full((1, _DEM_OUT)),
            full((_LIN_IN, _C)),
            full((1, _C)),
            full((_TLB, _TLB)),
        ],
        out_specs=[
            pl.BlockSpec(memory_space=pl.ANY),
            pl.BlockSpec((_B, _C, _TLB), lambda n1, j: (n1, 0, j)),
            pl.BlockSpec((_B, _C, _TLB), lambda n1, j: (n1, 0, j)),
        ],
        out_shape=[
            jax.ShapeDtypeStruct((_OUT_C, _NB1, _NJ, _B, _TLB), jnp.float32),
            jax.ShapeDtypeStruct((_N, _C, _L), jnp.int32),
            jax.ShapeDtypeStruct((_N, _C, _L), jnp.float32),
        ],
        scratch_shapes=[
            pltpu.VMEM((_B, _C), jnp.float32),
            pltpu.VMEM((_B, _C), jnp.float32),
            pltpu.VMEM((_B, _C), jnp.float32),
            pltpu.VMEM((2, _B, _OUT_C, _TLB), jnp.float32),
            pltpu.SemaphoreType.DMA((2,)),
        ],
        compiler_params=pltpu.CompilerParams(
            dimension_semantics=("arbitrary", "arbitrary"),
        ),
        name="deep_pool_encoder",
    )(dem, ts0, ts1, ts2, ts3,
      W_dem1, b_dem1.reshape(1, _DEM_H), W_dem2, b_dem2.reshape(1, _DEM_OUT),
      W_lin, b_lin.reshape(1, _C), tri)
    out = o5.transpose(1, 3, 2, 4, 0).reshape(_N, _L, _OUT_C)
    return out, ti, act


# speedup vs baseline: 19.8936x; 1.0012x over previous
import jax
import jax.numpy as jnp
from jax import lax
from jax.experimental import pallas as pl
from jax.experimental.pallas import tpu as pltpu

_N, _L = 32, 2048
_C = 256
_B = 8
_NB1 = _N // _B
_TLB = 128
_NJ = _L // _TLB
_NIT = _NB1 * _NJ
_DEM_IN, _DEM_H, _DEM_OUT = 8, 40, 20
_LIN_IN = 4 * _C + _DEM_OUT
_OUT_C = 3 * _C + _DEM_OUT
_EPS = 1e-5
_NEG = float("-inf")


def _sdown(v, k, fill):
    pad = jnp.full((_B, k, _C), fill, v.dtype)
    return jnp.concatenate([pad, v[:, : _TLB - k, :]], axis=1)


def _maxscan(v, fill):
    k = 1
    while k < _TLB:
        v = jnp.maximum(v, _sdown(v, k, fill))
        k *= 2
    return v


def _bdot(a, b):
    return jnp.dot(a.astype(jnp.bfloat16), b.astype(jnp.bfloat16),
                   preferred_element_type=jnp.float32)


def _body(dem_ref, ts0_ref, ts1_ref, ts2_ref, ts3_ref,
          wd1_ref, bd1_ref, wd2_ref, bd2_ref, wlin_ref, blin_ref, tri_ref,
          out_hbm, ti_ref, act_ref,
          cs_ref, cm_ref, ca_ref, dsc_ref, cdm_ref, obuf, osem):
    n1 = pl.program_id(0)
    j = pl.program_id(1)
    it = n1 * _NJ + j
    slot = lax.rem(it, 2)

    def owait(sl):
        for b in range(_B):
            pltpu.make_async_copy(obuf.at[sl, b], obuf.at[sl, b],
                                  osem.at[sl]).wait()

    @pl.when(it >= 2)
    def _reclaim():
        owait(slot)

    @pl.when(j == 0)
    def _init():
        cs_ref[...] = jnp.zeros_like(cs_ref)
        cm_ref[...] = jnp.full_like(cm_ref, _NEG)
        ca_ref[...] = jnp.full_like(ca_ref, -1.0)
        h = jnp.maximum(_bdot(dem_ref[...], wd1_ref[...]) + bd1_ref[...], 0.0)
        dall = jnp.maximum(_bdot(h, wd2_ref[...]) + bd2_ref[...], 0.0)
        d4 = dall.reshape(_NB1, _B, _DEM_OUT)
        gmask = lax.broadcasted_iota(jnp.int32, (_NB1, _B, _DEM_OUT), 0) == n1
        d8i = jnp.sum(jnp.where(gmask, d4, 0.0), axis=0)
        dsc_ref[...] = d8i
        cdm_ref[...] = _bdot(d8i, wlin_ref[4 * _C:, :]) + blin_ref[...]

    d8 = dsc_ref[...]

    t0 = ts0_ref[...].reshape(_B * _TLB, _C)
    t1 = ts1_ref[...].reshape(_B * _TLB, _C)
    t2 = ts2_ref[...].reshape(_B * _TLB, _C)
    t3 = ts3_ref[...].reshape(_B * _TLB, _C)
    xw = (_bdot(t0, wlin_ref[0 * _C:1 * _C, :])
          + _bdot(t1, wlin_ref[1 * _C:2 * _C, :])
          + _bdot(t2, wlin_ref[2 * _C:3 * _C, :])
          + _bdot(t3, wlin_ref[3 * _C:4 * _C, :]))
    x3 = xw.reshape(_B, _TLB, _C) + cdm_ref[...].reshape(_B, 1, _C)
    x2 = x3.reshape(_B * _TLB, _C)

    xhi = x2.astype(jnp.bfloat16)
    xlo = (x2 - xhi.astype(jnp.float32)).astype(jnp.bfloat16)
    trib = tri_ref[...]
    cs_c = cs_ref[...]
    chunks, carries = [], []
    for b in range(_B):
        sl = slice(b * _TLB, (b + 1) * _TLB)
        ck = (jnp.dot(trib, xhi[sl], preferred_element_type=jnp.float32)
              + jnp.dot(trib, xlo[sl], preferred_element_type=jnp.float32)
              + cs_c[b:b + 1, :])
        chunks.append(ck)
        carries.append(ck[_TLB - 1:_TLB, :])
    csum3 = jnp.concatenate(chunks, axis=0).reshape(_B, _TLB, _C)
    cs_ref[...] = jnp.concatenate(carries, axis=0)

    cmc = cm_ref[...].reshape(_B, 1, _C)
    m_loc = _maxscan(x3, _NEG)
    m_glob = jnp.maximum(m_loc, cmc)
    e = jnp.concatenate([cmc, m_glob[:, : _TLB - 1, :]], axis=1)
    cm_ref[...] = m_glob[:, _TLB - 1, :]

    tf = (j * _TLB
          + lax.broadcasted_iota(jnp.int32, (_B, _TLB, _C), 1)
          ).astype(jnp.float32)
    s = jnp.where(x3 > e, tf, -1.0)
    carg = jnp.maximum(_maxscan(s, -1.0), ca_ref[...].reshape(_B, 1, _C))
    ca_ref[...] = carg[:, _TLB - 1, :]

    csT = jnp.swapaxes(csum3, 1, 2)
    mT = jnp.swapaxes(m_glob, 1, 2)
    caT = jnp.swapaxes(carg, 1, 2)

    tT = j * _TLB + lax.broadcasted_iota(jnp.int32, (_B, _C, _TLB), 2)
    padT = tT < (_L - 1)
    pmaxT = jnp.where(padT, jnp.maximum(mT, 0.0), mT)
    tiT = jnp.where(padT & (pmaxT <= 0.0), tT - (_L - 1), caT.astype(jnp.int32))

    muc = jnp.mean(csT, axis=1, keepdims=True)
    varc = jnp.mean(csT * csT, axis=1, keepdims=True) - muc * muc
    dcT = csT - muc
    tlane = (j * _TLB
             + lax.broadcasted_iota(jnp.int32, (_B, 1, _TLB), 2)
             ).astype(jnp.float32)
    ln_avg = dcT * lax.rsqrt(varc + jnp.float32(_EPS * _L * _L))
    ln_sum = dcT * lax.rsqrt(varc + _EPS * (tlane + 1.0))

    mum = jnp.mean(pmaxT, axis=1, keepdims=True)
    varm = jnp.mean(pmaxT * pmaxT, axis=1, keepdims=True) - mum * mum
    ln_max = (pmaxT - mum) * lax.rsqrt(varm + _EPS)

    act_ref[...] = pmaxT
    ti_ref[...] = tiT

    obuf[slot, :, 0 * _C:1 * _C, :] = ln_max
    obuf[slot, :, 1 * _C:2 * _C, :] = ln_avg
    obuf[slot, :, 2 * _C:3 * _C, :] = ln_sum
    d8T = d8.T
    for b in range(_B):
        obuf[slot, b, 3 * _C:, :] = jnp.broadcast_to(
            d8T[:, b:b + 1], (_DEM_OUT, _TLB))
    for b in range(_B):
        pltpu.make_async_copy(obuf.at[slot, b],
                              out_hbm.at[:, n1, j, b, :],
                              osem.at[slot]).start()

    @pl.when(it == _NIT - 1)
    def _drain():
        owait(1 - slot)
        owait(slot)


def kernel(dem, ts0, ts1, ts2, ts3, W_dem1, b_dem1, W_dem2, b_dem2, W_lin, b_lin):
    tri = jnp.tril(jnp.ones((_TLB, _TLB), jnp.float32)).astype(jnp.bfloat16)
    full = lambda shape: pl.BlockSpec(shape, lambda n1, j: (0,) * len(shape))
    ts_spec = pl.BlockSpec((_B, _TLB, _C), lambda n1, j: (n1, j, 0))
    o5, ti, act = pl.pallas_call(
        _body,
        grid=(_NB1, _NJ),
        in_specs=[
            full((_N, _DEM_IN)),
            ts_spec, ts_spec, ts_spec, ts_spec,
            full((_DEM_IN, _DEM_H)),
            full((1, _DEM_H)),
            full((_DEM_H, _DEM_OUT)),
            full((1, _DEM_OUT)),
            full((_LIN_IN, _C)),
            full((1, _C)),
            full((_TLB, _TLB)),
        ],
        out_specs=[
            pl.BlockSpec(memory_space=pl.ANY),
            pl.BlockSpec((_B, _C, _TLB), lambda n1, j: (n1, 0, j)),
            pl.BlockSpec((_B, _C, _TLB), lambda n1, j: (n1, 0, j)),
        ],
        out_shape=[
            jax.ShapeDtypeStruct((_OUT_C, _NB1, _NJ, _B, _TLB), jnp.float32),
            jax.ShapeDtypeStruct((_N, _C, _L), jnp.int32),
            jax.ShapeDtypeStruct((_N, _C, _L), jnp.float32),
        ],
        scratch_shapes=[
            pltpu.VMEM((_B, _C), jnp.float32),
            pltpu.VMEM((_B, _C), jnp.float32),
            pltpu.VMEM((_B, _C), jnp.float32),
            pltpu.VMEM((_B, _DEM_OUT), jnp.float32),
            pltpu.VMEM((_B, _C), jnp.float32),
            pltpu.VMEM((2, _B, _OUT_C, _TLB), jnp.float32),
            pltpu.SemaphoreType.DMA((2,)),
        ],
        compiler_params=pltpu.CompilerParams(
            dimension_semantics=("arbitrary", "arbitrary"),
        ),
        name="deep_pool_encoder",
    )(dem, ts0, ts1, ts2, ts3,
      W_dem1, b_dem1.reshape(1, _DEM_H), W_dem2, b_dem2.reshape(1, _DEM_OUT),
      W_lin, b_lin.reshape(1, _C), tri)
    out = o5.transpose(1, 3, 2, 4, 0).reshape(_N, _L, _OUT_C)
    return out, ti, act
